# baseline (speedup 1.0000x reference)
import functools

import jax
import jax.numpy as jnp
from jax import lax
from jax.experimental import pallas as pl
from jax.experimental.pallas import tpu as pltpu
from jax.experimental.pallas import tpu_sc as plsc

S, E, K, D, B = 50000, 10000, 128, 128, 4096
NE = 500000

NC, NS, L = 2, 16, 16
NW = NC * NS
NEP = 524288
SPAD = 51200
EPAD = 10240
CH_BC = 2048
CH_RD = 256
DUMMY = 2048
ACC = B + DUMMY
EPT_BC = NEP // NW
EPT_RD = NEP // NW
SPT = SPAD // NS
EPT_STR = EPAD // NS
ACC_STR = ACC // NS
BPW = B // NW
RD_T0, RD_T1 = 16, 112
BC_T0, BC_T1 = 5, 11

_mesh = None


def _get_mesh():
    global _mesh
    if _mesh is None:
        _mesh = plsc.VectorSubcoreMesh(core_axis_name="c", subcore_axis_name="s")
    return _mesh


def _sc_bincount(eu1, ei1, eu0, ei0, zs, ze, ones):
    @functools.partial(
        pl.kernel, mesh=_get_mesh(),
        out_type=(
            jax.ShapeDtypeStruct((NC, SPAD), jnp.float32),
            jax.ShapeDtypeStruct((NC, SPAD), jnp.float32),
            jax.ShapeDtypeStruct((NC, EPAD), jnp.float32),
            jax.ShapeDtypeStruct((NC, EPAD), jnp.float32),
        ),
        scratch_types=[
            pltpu.VMEM((CH_BC,), jnp.int32),
            pltpu.VMEM((CH_BC,), jnp.float32),
            pltpu.VMEM_SHARED((SPAD,), jnp.float32),
            pltpu.VMEM_SHARED((EPAD,), jnp.float32),
            pltpu.SemaphoreType.DMA,
        ],
    )
    def k(eu1_h, ei1_h, eu0_h, ei0_h, zs_h, ze_h, ones_h,
          cu1_o, cu0_o, ci1_o, ci0_o,
          seg_v, ones_v, cnt_s, cnt_e, sem):
        c = lax.axis_index("c")
        s = lax.axis_index("s")
        w = c * NS + s
        pltpu.sync_copy(ones_h, ones_v)

        tcnt = jnp.where(c == 0, jnp.int32(BC_T0), jnp.int32(BC_T1))
        cbase = jnp.where(c == 0, s * BC_T0, NS * BC_T0 + s * BC_T1)

        def one_array(e_h, cnt, stripe, z_h, out):
            pltpu.sync_copy(z_h.at[pl.ds(s * stripe, stripe)],
                            cnt.at[pl.ds(s * stripe, stripe)])
            plsc.subcore_barrier()

            def bbody(i, carry):
                base = (cbase + i) * CH_BC
                pltpu.sync_copy(e_h.at[pl.ds(base, CH_BC)], seg_v)
                pltpu.sync_copy(ones_v, cnt.at[seg_v], add=True)
                return carry
            lax.fori_loop(0, tcnt, bbody, jnp.int32(0))
            plsc.subcore_barrier()
            pltpu.sync_copy(cnt.at[pl.ds(s * stripe, stripe)],
                            out.at[c].at[pl.ds(s * stripe, stripe)])
            plsc.subcore_barrier()

        one_array(eu1_h, cnt_s, SPT, zs_h, cu1_o)
        one_array(eu0_h, cnt_s, SPT, zs_h, cu0_o)
        one_array(ei1_h, cnt_e, EPT_STR, ze_h, ci1_o)
        one_array(ei0_h, cnt_e, EPT_STR, ze_h, ci0_o)

    return k(eu1, ei1, eu0, ei0, zs, ze, ones)


def _tc_tables(emb, ke, deg1, deg0, npad, blk):
    nblk = npad // blk

    def body(emb_r, ke_r, d1_r, d0_r, t1_r, t0_r):
        stat = jnp.dot(emb_r[...], ke_r[...].T, preferred_element_type=jnp.float32)
        s1 = jax.lax.rsqrt(d1_r[...] + 1.0)
        s0 = jax.lax.rsqrt(d0_r[...] + 1.0)
        t1_r[...] = stat * s1
        t0_r[...] = stat * s0

    return pl.pallas_call(
        body,
        grid=(nblk,),
        in_specs=[
            pl.BlockSpec((blk, K), lambda i: (i, 0)),
            pl.BlockSpec((K, K), lambda i: (0, 0)),
            pl.BlockSpec((blk, 1), lambda i: (i, 0)),
            pl.BlockSpec((blk, 1), lambda i: (i, 0)),
        ],
        out_specs=[
            pl.BlockSpec((blk, K), lambda i: (i, 0)),
            pl.BlockSpec((blk, K), lambda i: (i, 0)),
        ],
        out_shape=[
            jax.ShapeDtypeStruct((npad, K), jnp.float32),
            jax.ShapeDtypeStruct((npad, K), jnp.float32),
        ],
    )(emb, ke, deg1, deg0)


def _sc_reduce(eu1, ei1, eu0, ei0, pos_u, pos_e, a1, a0, b1, b0, zacc):
    out_sh = jax.ShapeDtypeStruct((NC, B, K), jnp.float32)
    NCHUNK = EPT_RD // CH_RD

    @functools.partial(
        pl.kernel, mesh=_get_mesh(),
        out_type=(out_sh, out_sh, out_sh, out_sh),
        scratch_types=[
            [pltpu.VMEM((CH_RD,), jnp.int32) for _ in range(2)],
            [pltpu.VMEM((CH_RD,), jnp.int32) for _ in range(2)],
            [pltpu.VMEM((CH_RD,), jnp.int32) for _ in range(2)],
            [pltpu.VMEM((CH_RD, K), jnp.float32) for _ in range(2)],
            pltpu.VMEM_SHARED((ACC, K), jnp.float32),
            [pltpu.SemaphoreType.DMA for _ in range(2)],
            [pltpu.SemaphoreType.DMA for _ in range(2)],
        ],
    )
    def k(eu1_h, ei1_h, eu0_h, ei0_h, pu_h, pe_h, a1_h, a0_h, b1_h, b0_h, z_h,
          u1_o, u0_o, i1_o, i0_o,
          dst_v, src_v, seg_v, rows_v, acc, sp, sr):
        c = lax.axis_index("c")
        s = lax.axis_index("s")
        w = c * NS + s

        T0, T1 = RD_T0, RD_T1
        tcnt = jnp.where(c == 0, jnp.int32(T0), jnp.int32(T1))
        cbase = jnp.where(c == 0, s * T0, NS * T0 + s * T1)

        def one_reduction(dst_h, src_h, pos_h, tab_h, out):
            pltpu.sync_copy(z_h, acc.at[pl.ds(s * ACC_STR, ACC_STR)])
            plsc.subcore_barrier()

            def load_and_fire(i, b):
                base = (cbase + i) * CH_RD
                pltpu.sync_copy(dst_h.at[pl.ds(base, CH_RD)], dst_v[b])
                pltpu.sync_copy(src_h.at[pl.ds(base, CH_RD)], src_v[b])
                pltpu.async_copy(pos_h.at[dst_v[b]], seg_v[b], sp[b])
                pltpu.async_copy(tab_h.at[src_v[b]], rows_v[b], sr[b])

            def consume(i, b):
                pltpu.make_async_copy(pos_h.at[dst_v[b]], seg_v[b], sp[b]).wait()
                for j in range(CH_RD // L):
                    v = seg_v[b][pl.ds(j * L, L)]
                    io = lax.iota(jnp.int32, L) + (
                        B + ((j * L) % DUMMY))
                    seg_v[b][pl.ds(j * L, L)] = jnp.where(v < B, v, io)
                pltpu.make_async_copy(tab_h.at[src_v[b]], rows_v[b], sr[b]).wait()
                pltpu.sync_copy(rows_v[b], acc.at[seg_v[b]], add=True)

            load_and_fire(0, 0)

            def body2(h, carry):
                i0i = h * 2

                @pl.when(i0i + 1 < tcnt)
                def _():
                    load_and_fire(i0i + 1, 1)
                consume(i0i, 0)

                @pl.when(i0i + 2 < tcnt)
                def _():
                    load_and_fire(i0i + 2, 0)

                @pl.when(i0i + 1 < tcnt)
                def _():
                    consume(i0i + 1, 1)
                return carry
            lax.fori_loop(0, (tcnt + 1) // 2, body2, jnp.int32(0))

            plsc.subcore_barrier()
            pltpu.sync_copy(acc.at[pl.ds(s * (B // NS), B // NS)],
                            out.at[c].at[pl.ds(s * (B // NS), B // NS)])
            plsc.subcore_barrier()

        one_reduction(eu1_h, ei1_h, pu_h, a1_h, u1_o)
        one_reduction(eu0_h, ei0_h, pu_h, a0_h, u0_o)
        one_reduction(ei1_h, eu1_h, pe_h, b1_h, i1_o)
        one_reduction(ei0_h, eu0_h, pe_h, b0_h, i0_o)

    return k(eu1, ei1, eu0, ei0, pos_u, pos_e, a1, a0, b1, b0, zacc)


def _sc_select(stu, exer, pos_u, pos_e, u1, u0, i1, i0, b1, a1,
               degu1, degu0, sebf, degi1, degi0, edf):
    row_sh = jax.ShapeDtypeStruct((B, K), jnp.float32)
    sca_sh = jax.ShapeDtypeStruct((B,), jnp.float32)

    @functools.partial(
        pl.kernel, mesh=_get_mesh(),
        out_type=(row_sh,) * 10 + (sca_sh,) * 6,
        scratch_types=[
            pltpu.VMEM((BPW,), jnp.int32),
            pltpu.VMEM((BPW,), jnp.int32),
            pltpu.VMEM((BPW,), jnp.int32),
            pltpu.VMEM((BPW,), jnp.int32),
            pltpu.VMEM((BPW, K), jnp.float32),
            pltpu.VMEM((BPW,), jnp.float32),
            pltpu.SemaphoreType.DMA,
        ],
    )
    def k(stu_h, exer_h, pu_h, pe_h, u1_h, u0_h, i1_h, i0_h, b1_h, a1_h,
          degu1_h, degu0_h, sebf_h, degi1_h, degi0_h, edf_h,
          u1a_o, u1b_o, u0a_o, u0b_o, i1a_o, i1b_o, i0a_o, i0b_o, rb1_o, ra1_o,
          du1_o, du0_o, seb_o, di1_o, di0_o, ed_o,
          stu_v, exer_v, slu_v, sle_v, buf_v, f1_v, sem):
        c = lax.axis_index("c")
        s = lax.axis_index("s")
        w = c * NS + s
        base = w * BPW
        pltpu.sync_copy(stu_h.at[pl.ds(base, BPW)], stu_v)
        pltpu.sync_copy(exer_h.at[pl.ds(base, BPW)], exer_v)
        pltpu.async_copy(pu_h.at[stu_v], slu_v, sem).wait()
        pltpu.async_copy(pe_h.at[exer_v], sle_v, sem).wait()

        def grab(tab_h, idx_v, out):
            pltpu.async_copy(tab_h.at[idx_v], buf_v, sem).wait()
            pltpu.sync_copy(buf_v, out.at[pl.ds(base, BPW)])

        def grab_scalar(tab_h, idx_v, out):
            pltpu.async_copy(tab_h.at[idx_v], f1_v, sem).wait()
            pltpu.sync_copy(f1_v, out.at[pl.ds(base, BPW)])

        grab(u1_h, slu_v, u1a_o)
        grab(u0_h, slu_v, u0a_o)
        grab(i1_h, sle_v, i1a_o)
        grab(i0_h, sle_v, i0a_o)
        for j in range(BPW // L):
            slu_v[pl.ds(j * L, L)] = slu_v[pl.ds(j * L, L)] + B
            sle_v[pl.ds(j * L, L)] = sle_v[pl.ds(j * L, L)] + B
        grab(u1_h, slu_v, u1b_o)
        grab(u0_h, slu_v, u0b_o)
        grab(i1_h, sle_v, i1b_o)
        grab(i0_h, sle_v, i0b_o)
        grab(b1_h, stu_v, rb1_o)
        grab(a1_h, exer_v, ra1_o)

        grab_scalar(degu1_h, stu_v, du1_o)
        grab_scalar(degu0_h, stu_v, du0_o)
        grab_scalar(sebf_h, stu_v, seb_o)
        grab_scalar(degi1_h, exer_v, di1_o)
        grab_scalar(degi0_h, exer_v, di0_o)
        grab_scalar(edf_h, exer_v, ed_o)

    return k(stu, exer, pos_u, pos_e, u1, u0, i1, i0, b1, a1,
             degu1, degu0, sebf, degi1, degi0, edf)


def _tc_final(u1a, u1b, u0a, u0b, i1a, i1b, i0a, i0b, rb1, ra1,
              du1, du0, seb_g, di1, di0, ed_g,
              ikp, w1w, w1b, w0w, w0b, p1w, p1b, p2w, p2b, p3w, p3b):
    blk = 512
    nblk = B // blk

    def body(u1a_r, u1b_r, u0a_r, u0b_r, i1a_r, i1b_r, i0a_r, i0b_r,
             rb1_r, ra1_r, du1_r, du0_r, seb_r, di1_r, di0_r, ed_r, ikp_r,
             w1w_r, w1b_r, w0w_r, w0b_r, p1w_r, p1b_r, p2w_r, p2b_r,
             p3w_r, p3b_r, out_r):
        d1 = 1.0 / (du1_r[...] + 1.0)
        d0 = 1.0 / (du0_r[...] + 1.0)
        seb = seb_r[...]
        s1 = jnp.sqrt(d1)
        s0 = jnp.sqrt(d0)
        e1 = 1.0 / (di1_r[...] + 1.0)
        e0 = 1.0 / (di0_r[...] + 1.0)
        ed = ed_r[...]
        t1 = jnp.sqrt(e1)
        t0 = jnp.sqrt(e0)
        stat = rb1_r[...] / s1
        kd = ra1_r[...] / t1
        gu1 = s1 * (u1a_r[...] + u1b_r[...]) + d1 * stat
        gu0 = s0 * (u0a_r[...] + u0b_r[...]) + d0 * stat
        gi1 = t1 * (i1a_r[...] + i1b_r[...]) + e1 * kd
        gi0 = t0 * (i0a_r[...] + i0b_r[...]) + e0 * kd
        w1 = w1w_r[...]
        w0 = w0w_r[...]
        stat_f = (jnp.dot(gu1, w1.T, preferred_element_type=jnp.float32) + w1b_r[...]
                  + jnp.dot(gu0, w0.T, preferred_element_type=jnp.float32) + w0b_r[...])
        diff_f = (jnp.dot(gi1, w1.T, preferred_element_type=jnp.float32) + w1b_r[...]
                  + jnp.dot(gi0, w0.T, preferred_element_type=jnp.float32) + w0b_r[...])
        stat_b = jax.nn.sigmoid(stat_f + seb)
        diff_b = jax.nn.sigmoid(diff_f)
        disc = jax.nn.sigmoid(ed) * 10.0
        x = disc * (stat_b - diff_b) * ikp_r[...]
        h = jax.nn.sigmoid(
            jnp.dot(x, jnp.abs(p1w_r[...]).T, preferred_element_type=jnp.float32)
            + p1b_r[...])
        h = jax.nn.sigmoid(
            jnp.dot(h, jnp.abs(p2w_r[...]).T, preferred_element_type=jnp.float32)
            + p2b_r[...])
        o = jax.nn.sigmoid(
            jnp.sum(h * jnp.abs(p3w_r[...]), axis=1, keepdims=True)
            + p3b_r[...])
        out_r[...] = o

    full = lambda shape: pl.BlockSpec(shape, lambda i: tuple(0 for _ in shape))
    rows = pl.BlockSpec((blk, K), lambda i: (i, 0))
    sca = pl.BlockSpec((blk, 1), lambda i: (i, 0))
    return pl.pallas_call(
        body,
        grid=(nblk,),
        in_specs=[rows] * 10 + [sca] * 6 + [rows,
                                full((K, K)), full((1, K)), full((K, K)), full((1, K)),
                                full((256, K)), full((1, 256)), full((K, 256)), full((1, K)),
                                full((1, K)), full((1, 1))],
        out_specs=pl.BlockSpec((blk, 1), lambda i: (i, 0)),
        out_shape=jax.ShapeDtypeStruct((B, 1), jnp.float32),
    )(u1a, u1b, u0a, u0b, i1a, i1b, i0a, i0b, rb1, ra1,
      du1.reshape(B, 1), du0.reshape(B, 1), seb_g.reshape(B, 1),
      di1.reshape(B, 1), di0.reshape(B, 1), ed_g.reshape(B, 1),
      ikp, w1w, w1b, w0w, w0b, p1w, p1b, p2w, p2b, p3w, p3b)


def kernel(stu_id, input_exercise, input_knowledge_point, edge_u_1, edge_i_1,
           edge_u_0, edge_i_0, student_emb, student_emb_bias, exercise_emb,
           knowledge_emb, e_discrimination, W1_w, W1_b, W0_w, W0_b,
           p1_w, p1_b, p2_w, p2_b, p3_w, p3_b):
    i32 = jnp.int32
    stu = stu_id.astype(i32)
    exer = input_exercise.astype(i32)

    def pad_e(e, fill):
        return jnp.concatenate(
            [e.astype(i32), jnp.full((NEP - NE,), fill, i32)])
    eu1 = pad_e(edge_u_1, SPAD - 1)
    eu0 = pad_e(edge_u_0, SPAD - 1)
    ei1 = pad_e(edge_i_1, EPAD - 1)
    ei0 = pad_e(edge_i_0, EPAD - 1)

    pos_u = jnp.full((SPAD,), B, i32).at[stu].set(jnp.arange(B, dtype=i32))
    pos_e = jnp.full((EPAD,), B, i32).at[exer].set(jnp.arange(B, dtype=i32))

    zs = jnp.zeros((SPAD,), jnp.float32)
    ze = jnp.zeros((EPAD,), jnp.float32)
    ones = jnp.ones((CH_BC,), jnp.float32)

    cu1, cu0, ci1, ci0 = _sc_bincount(eu1, ei1, eu0, ei0, zs, ze, ones)

    sep = jnp.pad(student_emb, ((0, SPAD - S), (0, 0)))
    sebf = jnp.pad(student_emb_bias.reshape(-1), (0, SPAD - S))
    eep = jnp.pad(exercise_emb, ((0, EPAD - E), (0, 0)))
    edf = jnp.pad(e_discrimination.reshape(-1), (0, EPAD - E))

    degu1 = cu1[0] + cu1[1]
    degu0 = cu0[0] + cu0[1]
    degi1 = ci1[0] + ci1[1]
    degi0 = ci0[0] + ci0[1]

    b1, b0 = _tc_tables(sep, knowledge_emb, degu1.reshape(SPAD, 1),
                        degu0.reshape(SPAD, 1), SPAD, 512)
    a1, a0 = _tc_tables(eep, knowledge_emb, degi1.reshape(EPAD, 1),
                        degi0.reshape(EPAD, 1), EPAD, 512)

    zacc = jnp.zeros((ACC_STR, K), jnp.float32)
    u1, u0, i1, i0 = _sc_reduce(eu1, ei1, eu0, ei0, pos_u, pos_e,
                                a1, a0, b1, b0, zacc)

    flat = lambda t: t.reshape(NC * B, K)
    (u1a, u1b, u0a, u0b, i1a, i1b, i0a, i0b, rb1, ra1,
     du1, du0, seb_g, di1, di0, ed_g) = _sc_select(
        stu, exer, pos_u, pos_e, flat(u1), flat(u0), flat(i1), flat(i0),
        b1, a1, degu1, degu0, sebf, degi1, degi0, edf)

    out = _tc_final(u1a, u1b, u0a, u0b, i1a, i1b, i0a, i0b, rb1, ra1,
                    du1, du0, seb_g, di1, di0, ed_g,
                    input_knowledge_point,
                    W1_w, W1_b.reshape(1, K), W0_w, W0_b.reshape(1, K),
                    p1_w, p1_b.reshape(1, 256), p2_w, p2_b.reshape(1, K),
                    p3_w, p3_b.reshape(1, 1))
    return out.reshape(-1)


# speedup vs baseline: 1.0435x; 1.0435x over previous
import functools

import jax
import jax.numpy as jnp
from jax import lax
from jax.experimental import pallas as pl
from jax.experimental.pallas import tpu as pltpu
from jax.experimental.pallas import tpu_sc as plsc

S, E, K, D, B = 50000, 10000, 128, 128, 4096
NE = 500000

NC, NS, L = 2, 16, 16
NW = NC * NS
NEP = 524288
SPAD = 51200
EPAD = 10240
CH_BC = 2048
CH_RD = 256
DUMMY = 2048
ACC = B + DUMMY
EPT_BC = NEP // NW
EPT_RD = NEP // NW
SPT = SPAD // NS
EPT_STR = EPAD // NS
ACC_STR = ACC // NS
BPW = B // NW
RD_T0, RD_T1 = 64, 64
BC_T0, BC_T1 = 8, 8

_mesh = None


def _get_mesh():
    global _mesh
    if _mesh is None:
        _mesh = plsc.VectorSubcoreMesh(core_axis_name="c", subcore_axis_name="s")
    return _mesh


def _sc_bincount(eu1, ei1, eu0, ei0, zs, ze, ones):
    @functools.partial(
        pl.kernel, mesh=_get_mesh(),
        out_type=(
            jax.ShapeDtypeStruct((NC, SPAD), jnp.float32),
            jax.ShapeDtypeStruct((NC, SPAD), jnp.float32),
            jax.ShapeDtypeStruct((NC, EPAD), jnp.float32),
            jax.ShapeDtypeStruct((NC, EPAD), jnp.float32),
        ),
        scratch_types=[
            pltpu.VMEM((CH_BC,), jnp.int32),
            pltpu.VMEM((CH_BC,), jnp.float32),
            pltpu.VMEM_SHARED((SPAD,), jnp.float32),
            pltpu.VMEM_SHARED((EPAD,), jnp.float32),
            pltpu.SemaphoreType.DMA,
        ],
    )
    def k(eu1_h, ei1_h, eu0_h, ei0_h, zs_h, ze_h, ones_h,
          cu1_o, cu0_o, ci1_o, ci0_o,
          seg_v, ones_v, cnt_s, cnt_e, sem):
        c = lax.axis_index("c")
        s = lax.axis_index("s")
        w = c * NS + s
        pltpu.sync_copy(ones_h, ones_v)

        tcnt = jnp.where(c == 0, jnp.int32(BC_T0), jnp.int32(BC_T1))
        cbase = jnp.where(c == 0, s * BC_T0, NS * BC_T0 + s * BC_T1)

        def one_array(e_h, cnt, stripe, z_h, out):
            pltpu.sync_copy(z_h.at[pl.ds(s * stripe, stripe)],
                            cnt.at[pl.ds(s * stripe, stripe)])
            plsc.subcore_barrier()

            def bbody(i, carry):
                base = (cbase + i) * CH_BC
                pltpu.sync_copy(e_h.at[pl.ds(base, CH_BC)], seg_v)
                pltpu.sync_copy(ones_v, cnt.at[seg_v], add=True)
                return carry
            lax.fori_loop(0, tcnt, bbody, jnp.int32(0))
            plsc.subcore_barrier()
            pltpu.sync_copy(cnt.at[pl.ds(s * stripe, stripe)],
                            out.at[c].at[pl.ds(s * stripe, stripe)])
            plsc.subcore_barrier()

        one_array(eu1_h, cnt_s, SPT, zs_h, cu1_o)
        one_array(eu0_h, cnt_s, SPT, zs_h, cu0_o)
        one_array(ei1_h, cnt_e, EPT_STR, ze_h, ci1_o)
        one_array(ei0_h, cnt_e, EPT_STR, ze_h, ci0_o)

    return k(eu1, ei1, eu0, ei0, zs, ze, ones)


def _tc_tables(emb, ke, deg1, deg0, npad, blk):
    nblk = npad // blk

    def body(emb_r, ke_r, d1_r, d0_r, t1_r, t0_r):
        stat = jnp.dot(emb_r[...], ke_r[...].T, preferred_element_type=jnp.float32)
        s1 = jax.lax.rsqrt(d1_r[...] + 1.0)
        s0 = jax.lax.rsqrt(d0_r[...] + 1.0)
        t1_r[...] = stat * s1
        t0_r[...] = stat * s0

    return pl.pallas_call(
        body,
        grid=(nblk,),
        in_specs=[
            pl.BlockSpec((blk, K), lambda i: (i, 0)),
            pl.BlockSpec((K, K), lambda i: (0, 0)),
            pl.BlockSpec((blk, 1), lambda i: (i, 0)),
            pl.BlockSpec((blk, 1), lambda i: (i, 0)),
        ],
        out_specs=[
            pl.BlockSpec((blk, K), lambda i: (i, 0)),
            pl.BlockSpec((blk, K), lambda i: (i, 0)),
        ],
        out_shape=[
            jax.ShapeDtypeStruct((npad, K), jnp.float32),
            jax.ShapeDtypeStruct((npad, K), jnp.float32),
        ],
    )(emb, ke, deg1, deg0)


def _sc_reduce(eu1, ei1, eu0, ei0, pos_u, pos_e, a1, a0, b1, b0, zacc):
    out_sh = jax.ShapeDtypeStruct((NC, B, K), jnp.float32)
    NCHUNK = EPT_RD // CH_RD

    @functools.partial(
        pl.kernel, mesh=_get_mesh(),
        out_type=(out_sh, out_sh, out_sh, out_sh),
        scratch_types=[
            [pltpu.VMEM((CH_RD,), jnp.int32) for _ in range(2)],
            [pltpu.VMEM((CH_RD,), jnp.int32) for _ in range(2)],
            [pltpu.VMEM((CH_RD,), jnp.int32) for _ in range(2)],
            [pltpu.VMEM((CH_RD, K), jnp.float32) for _ in range(2)],
            pltpu.VMEM((96, K), jnp.float32),
            pltpu.VMEM_SHARED((ACC, K), jnp.float32),
            [pltpu.SemaphoreType.DMA for _ in range(2)],
            [pltpu.SemaphoreType.DMA for _ in range(2)],
        ],
    )
    def k(eu1_h, ei1_h, eu0_h, ei0_h, pu_h, pe_h, a1_h, a0_h, b1_h, b0_h, z_h,
          u1_o, u0_o, i1_o, i0_o,
          dst_v, src_v, seg_v, rows_v, zbuf_v, acc, sp, sr):
        c = lax.axis_index("c")
        s = lax.axis_index("s")
        w = c * NS + s

        T0, T1 = RD_T0, RD_T1
        tcnt = jnp.where(c == 0, jnp.int32(T0), jnp.int32(T1))
        cbase = jnp.where(c == 0, s * T0, NS * T0 + s * T1)

        zrows = jnp.zeros((L,), jnp.float32)
        for zi in range(96):
            for zj in range(K // L):
                zbuf_v[zi, pl.ds(zj * L, L)] = zrows

        def one_reduction(dst_h, src_h, pos_h, tab_h, out):
            for zr in range(ACC_STR // 96):
                pltpu.sync_copy(zbuf_v,
                                acc.at[pl.ds(s * ACC_STR + zr * 96, 96)])
            plsc.subcore_barrier()

            def load_and_fire(i, b):
                base = (cbase + i) * CH_RD
                pltpu.sync_copy(dst_h.at[pl.ds(base, CH_RD)], dst_v[b])
                pltpu.sync_copy(src_h.at[pl.ds(base, CH_RD)], src_v[b])
                pltpu.async_copy(pos_h.at[dst_v[b]], seg_v[b], sp[b])
                pltpu.async_copy(tab_h.at[src_v[b]], rows_v[b], sr[b])

            def consume(i, b):
                pltpu.make_async_copy(pos_h.at[dst_v[b]], seg_v[b], sp[b]).wait()
                for j in range(CH_RD // L):
                    v = seg_v[b][pl.ds(j * L, L)]
                    io = lax.iota(jnp.int32, L) + (
                        B + ((j * L) % DUMMY))
                    seg_v[b][pl.ds(j * L, L)] = jnp.where(v < B, v, io)
                pltpu.make_async_copy(tab_h.at[src_v[b]], rows_v[b], sr[b]).wait()
                pltpu.sync_copy(rows_v[b], acc.at[seg_v[b]], add=True)

            load_and_fire(0, 0)

            def body2(h, carry):
                i0i = h * 2

                @pl.when(i0i + 1 < tcnt)
                def _():
                    load_and_fire(i0i + 1, 1)
                consume(i0i, 0)

                @pl.when(i0i + 2 < tcnt)
                def _():
                    load_and_fire(i0i + 2, 0)

                @pl.when(i0i + 1 < tcnt)
                def _():
                    consume(i0i + 1, 1)
                return carry
            lax.fori_loop(0, (tcnt + 1) // 2, body2, jnp.int32(0))

            plsc.subcore_barrier()
            pltpu.sync_copy(acc.at[pl.ds(s * (B // NS), B // NS)],
                            out.at[c].at[pl.ds(s * (B // NS), B // NS)])
            plsc.subcore_barrier()

        one_reduction(eu1_h, ei1_h, pu_h, a1_h, u1_o)
        one_reduction(eu0_h, ei0_h, pu_h, a0_h, u0_o)
        one_reduction(ei1_h, eu1_h, pe_h, b1_h, i1_o)
        one_reduction(ei0_h, eu0_h, pe_h, b0_h, i0_o)

    return k(eu1, ei1, eu0, ei0, pos_u, pos_e, a1, a0, b1, b0, zacc)


def _sc_select(stu, exer, pos_u, pos_e, u1, u0, i1, i0, b1, a1,
               degu1, degu0, sebf, degi1, degi0, edf):
    row_sh = jax.ShapeDtypeStruct((B, K), jnp.float32)
    sca_sh = jax.ShapeDtypeStruct((B,), jnp.float32)

    @functools.partial(
        pl.kernel, mesh=_get_mesh(),
        out_type=(row_sh,) * 10 + (sca_sh,) * 6,
        scratch_types=[
            pltpu.VMEM((BPW,), jnp.int32),
            pltpu.VMEM((BPW,), jnp.int32),
            pltpu.VMEM((BPW,), jnp.int32),
            pltpu.VMEM((BPW,), jnp.int32),
            pltpu.VMEM((BPW, K), jnp.float32),
            pltpu.VMEM((BPW,), jnp.float32),
            pltpu.SemaphoreType.DMA,
        ],
    )
    def k(stu_h, exer_h, pu_h, pe_h, u1_h, u0_h, i1_h, i0_h, b1_h, a1_h,
          degu1_h, degu0_h, sebf_h, degi1_h, degi0_h, edf_h,
          u1a_o, u1b_o, u0a_o, u0b_o, i1a_o, i1b_o, i0a_o, i0b_o, rb1_o, ra1_o,
          du1_o, du0_o, seb_o, di1_o, di0_o, ed_o,
          stu_v, exer_v, slu_v, sle_v, buf_v, f1_v, sem):
        c = lax.axis_index("c")
        s = lax.axis_index("s")
        w = c * NS + s
        base = w * BPW
        pltpu.sync_copy(stu_h.at[pl.ds(base, BPW)], stu_v)
        pltpu.sync_copy(exer_h.at[pl.ds(base, BPW)], exer_v)
        pltpu.async_copy(pu_h.at[stu_v], slu_v, sem).wait()
        pltpu.async_copy(pe_h.at[exer_v], sle_v, sem).wait()

        def grab(tab_h, idx_v, out):
            pltpu.async_copy(tab_h.at[idx_v], buf_v, sem).wait()
            pltpu.sync_copy(buf_v, out.at[pl.ds(base, BPW)])

        def grab_scalar(tab_h, idx_v, out):
            pltpu.async_copy(tab_h.at[idx_v], f1_v, sem).wait()
            pltpu.sync_copy(f1_v, out.at[pl.ds(base, BPW)])

        grab(u1_h, slu_v, u1a_o)
        grab(u0_h, slu_v, u0a_o)
        grab(i1_h, sle_v, i1a_o)
        grab(i0_h, sle_v, i0a_o)
        for j in range(BPW // L):
            slu_v[pl.ds(j * L, L)] = slu_v[pl.ds(j * L, L)] + B
            sle_v[pl.ds(j * L, L)] = sle_v[pl.ds(j * L, L)] + B
        grab(u1_h, slu_v, u1b_o)
        grab(u0_h, slu_v, u0b_o)
        grab(i1_h, sle_v, i1b_o)
        grab(i0_h, sle_v, i0b_o)
        grab(b1_h, stu_v, rb1_o)
        grab(a1_h, exer_v, ra1_o)

        grab_scalar(degu1_h, stu_v, du1_o)
        grab_scalar(degu0_h, stu_v, du0_o)
        grab_scalar(sebf_h, stu_v, seb_o)
        grab_scalar(degi1_h, exer_v, di1_o)
        grab_scalar(degi0_h, exer_v, di0_o)
        grab_scalar(edf_h, exer_v, ed_o)

    return k(stu, exer, pos_u, pos_e, u1, u0, i1, i0, b1, a1,
             degu1, degu0, sebf, degi1, degi0, edf)


def _tc_final(u1a, u1b, u0a, u0b, i1a, i1b, i0a, i0b, rb1, ra1,
              du1, du0, seb_g, di1, di0, ed_g,
              ikp, w1w, w1b, w0w, w0b, p1w, p1b, p2w, p2b, p3w, p3b):
    blk = 512
    nblk = B // blk

    def body(u1a_r, u1b_r, u0a_r, u0b_r, i1a_r, i1b_r, i0a_r, i0b_r,
             rb1_r, ra1_r, du1_r, du0_r, seb_r, di1_r, di0_r, ed_r, ikp_r,
             w1w_r, w1b_r, w0w_r, w0b_r, p1w_r, p1b_r, p2w_r, p2b_r,
             p3w_r, p3b_r, out_r):
        d1 = 1.0 / (du1_r[...] + 1.0)
        d0 = 1.0 / (du0_r[...] + 1.0)
        seb = seb_r[...]
        s1 = jnp.sqrt(d1)
        s0 = jnp.sqrt(d0)
        e1 = 1.0 / (di1_r[...] + 1.0)
        e0 = 1.0 / (di0_r[...] + 1.0)
        ed = ed_r[...]
        t1 = jnp.sqrt(e1)
        t0 = jnp.sqrt(e0)
        stat = rb1_r[...] / s1
        kd = ra1_r[...] / t1
        gu1 = s1 * (u1a_r[...] + u1b_r[...]) + d1 * stat
        gu0 = s0 * (u0a_r[...] + u0b_r[...]) + d0 * stat
        gi1 = t1 * (i1a_r[...] + i1b_r[...]) + e1 * kd
        gi0 = t0 * (i0a_r[...] + i0b_r[...]) + e0 * kd
        w1 = w1w_r[...]
        w0 = w0w_r[...]
        stat_f = (jnp.dot(gu1, w1.T, preferred_element_type=jnp.float32) + w1b_r[...]
                  + jnp.dot(gu0, w0.T, preferred_element_type=jnp.float32) + w0b_r[...])
        diff_f = (jnp.dot(gi1, w1.T, preferred_element_type=jnp.float32) + w1b_r[...]
                  + jnp.dot(gi0, w0.T, preferred_element_type=jnp.float32) + w0b_r[...])
        stat_b = jax.nn.sigmoid(stat_f + seb)
        diff_b = jax.nn.sigmoid(diff_f)
        disc = jax.nn.sigmoid(ed) * 10.0
        x = disc * (stat_b - diff_b) * ikp_r[...]
        h = jax.nn.sigmoid(
            jnp.dot(x, jnp.abs(p1w_r[...]).T, preferred_element_type=jnp.float32)
            + p1b_r[...])
        h = jax.nn.sigmoid(
            jnp.dot(h, jnp.abs(p2w_r[...]).T, preferred_element_type=jnp.float32)
            + p2b_r[...])
        o = jax.nn.sigmoid(
            jnp.sum(h * jnp.abs(p3w_r[...]), axis=1, keepdims=True)
            + p3b_r[...])
        out_r[...] = o

    full = lambda shape: pl.BlockSpec(shape, lambda i: tuple(0 for _ in shape))
    rows = pl.BlockSpec((blk, K), lambda i: (i, 0))
    sca = pl.BlockSpec((blk, 1), lambda i: (i, 0))
    return pl.pallas_call(
        body,
        grid=(nblk,),
        in_specs=[rows] * 10 + [sca] * 6 + [rows,
                                full((K, K)), full((1, K)), full((K, K)), full((1, K)),
                                full((256, K)), full((1, 256)), full((K, 256)), full((1, K)),
                                full((1, K)), full((1, 1))],
        out_specs=pl.BlockSpec((blk, 1), lambda i: (i, 0)),
        out_shape=jax.ShapeDtypeStruct((B, 1), jnp.float32),
    )(u1a, u1b, u0a, u0b, i1a, i1b, i0a, i0b, rb1, ra1,
      du1.reshape(B, 1), du0.reshape(B, 1), seb_g.reshape(B, 1),
      di1.reshape(B, 1), di0.reshape(B, 1), ed_g.reshape(B, 1),
      ikp, w1w, w1b, w0w, w0b, p1w, p1b, p2w, p2b, p3w, p3b)


def kernel(stu_id, input_exercise, input_knowledge_point, edge_u_1, edge_i_1,
           edge_u_0, edge_i_0, student_emb, student_emb_bias, exercise_emb,
           knowledge_emb, e_discrimination, W1_w, W1_b, W0_w, W0_b,
           p1_w, p1_b, p2_w, p2_b, p3_w, p3_b):
    i32 = jnp.int32
    stu = stu_id.astype(i32)
    exer = input_exercise.astype(i32)

    def pad_e(e, fill):
        return jnp.concatenate(
            [e.astype(i32), jnp.full((NEP - NE,), fill, i32)])
    eu1 = pad_e(edge_u_1, SPAD - 1)
    eu0 = pad_e(edge_u_0, SPAD - 1)
    ei1 = pad_e(edge_i_1, EPAD - 1)
    ei0 = pad_e(edge_i_0, EPAD - 1)

    pos_u = jnp.full((SPAD,), B, i32).at[stu].set(jnp.arange(B, dtype=i32))
    pos_e = jnp.full((EPAD,), B, i32).at[exer].set(jnp.arange(B, dtype=i32))

    zs = jnp.zeros((SPAD,), jnp.float32)
    ze = jnp.zeros((EPAD,), jnp.float32)
    ones = jnp.ones((CH_BC,), jnp.float32)

    cu1, cu0, ci1, ci0 = _sc_bincount(eu1, ei1, eu0, ei0, zs, ze, ones)

    sep = jnp.pad(student_emb, ((0, SPAD - S), (0, 0)))
    sebf = jnp.pad(student_emb_bias.reshape(-1), (0, SPAD - S))
    eep = jnp.pad(exercise_emb, ((0, EPAD - E), (0, 0)))
    edf = jnp.pad(e_discrimination.reshape(-1), (0, EPAD - E))

    degu1 = cu1[0] + cu1[1]
    degu0 = cu0[0] + cu0[1]
    degi1 = ci1[0] + ci1[1]
    degi0 = ci0[0] + ci0[1]

    b1, b0 = _tc_tables(sep, knowledge_emb, degu1.reshape(SPAD, 1),
                        degu0.reshape(SPAD, 1), SPAD, 512)
    a1, a0 = _tc_tables(eep, knowledge_emb, degi1.reshape(EPAD, 1),
                        degi0.reshape(EPAD, 1), EPAD, 512)

    zacc = jnp.zeros((ACC_STR, K), jnp.float32)
    u1, u0, i1, i0 = _sc_reduce(eu1, ei1, eu0, ei0, pos_u, pos_e,
                                a1, a0, b1, b0, zacc)

    flat = lambda t: t.reshape(NC * B, K)
    (u1a, u1b, u0a, u0b, i1a, i1b, i0a, i0b, rb1, ra1,
     du1, du0, seb_g, di1, di0, ed_g) = _sc_select(
        stu, exer, pos_u, pos_e, flat(u1), flat(u0), flat(i1), flat(i0),
        b1, a1, degu1, degu0, sebf, degi1, degi0, edf)

    out = _tc_final(u1a, u1b, u0a, u0b, i1a, i1b, i0a, i0b, rb1, ra1,
                    du1, du0, seb_g, di1, di0, ed_g,
                    input_knowledge_point,
                    W1_w, W1_b.reshape(1, K), W0_w, W0_b.reshape(1, K),
                    p1_w, p1_b.reshape(1, 256), p2_w, p2_b.reshape(1, K),
                    p3_w, p3_b.reshape(1, 1))
    return out.reshape(-1)
